# baseline (speedup 1.0000x reference)
import functools

import jax
import jax.numpy as jnp
from jax import lax
from jax.experimental import pallas as pl
from jax.experimental.pallas import tpu as pltpu
from jax.experimental.pallas import tpu_sc as plsc

NC = 2
NS = 16
NW = NC * NS
L = 16

C = 8192
PAD = 16
CU_T = 528


@functools.lru_cache(maxsize=None)
def _build(n, n_rays):
    rpw = n_rays // NW
    mesh = plsc.VectorSubcoreMesh(core_axis_name="c", subcore_axis_name="s",
                                  num_cores=NC, num_subcores=NS)

    @functools.partial(
        pl.kernel,
        mesh=mesh,
        compiler_params=pltpu.CompilerParams(needs_layout_passes=False),
        out_type=[
            jax.ShapeDtypeStruct((n_rays,), jnp.float32),
            jax.ShapeDtypeStruct((n_rays,), jnp.float32),
            jax.ShapeDtypeStruct((3 * n_rays,), jnp.float32),
        ],
        scratch_types=[
            pltpu.VMEM((2 * C,), jnp.float32),
            pltpu.VMEM((2 * C,), jnp.float32),
            pltpu.VMEM((2 * (C + PAD),), jnp.float32),
            pltpu.VMEM((2 * (C + PAD),), jnp.float32),
            pltpu.VMEM((2 * (C + PAD),), jnp.float32),
            pltpu.VMEM((C + PAD,), jnp.float32),
            pltpu.VMEM((C + PAD,), jnp.float32),
            pltpu.VMEM((CU_T,), jnp.int32),
            pltpu.VMEM((rpw,), jnp.float32),
            pltpu.VMEM((rpw,), jnp.float32),
            pltpu.VMEM((3 * rpw,), jnp.float32),
            pltpu.SemaphoreType.DMA,
            pltpu.SemaphoreType.DMA,
        ],
    )
    def k(dt_h, dens_h, r_h, g_h, b_h, cu_h, ws_h, bg_h, rgbo_h,
          dtb, dnb, rb, gb, bb, albuf, pbuf, cuv, ows, obg, orgb,
          semA, semB):
        wid = lax.axis_index("s") * NC + lax.axis_index("c")
        r0 = pl.multiple_of(wid * rpw, 8)
        pltpu.sync_copy(cu_h.at[pl.ds(r0, CU_T)], cuv)
        iota = lax.iota(jnp.int32, L)
        lane0 = iota == 0
        fz = jnp.zeros((L,), jnp.float32)
        zero = jnp.float32(0.0)

        def sread(ref, i):
            return plsc.load_gather(ref, [jnp.full((L,), i, jnp.int32)])[0]

        def issue(b, row):
            oc = pl.multiple_of(row * C, 8)
            op = pl.multiple_of(row * (C + PAD), 8)
            pltpu.async_copy(dt_h.at[pl.ds(b, C)], dtb.at[pl.ds(oc, C)],
                             semA)
            pltpu.async_copy(dens_h.at[pl.ds(b, C)], dnb.at[pl.ds(oc, C)],
                             semA)
            pltpu.async_copy(r_h.at[pl.ds(b, C)], rb.at[pl.ds(op, C)], semB)
            pltpu.async_copy(g_h.at[pl.ds(b, C)], gb.at[pl.ds(op, C)], semB)
            pltpu.async_copy(b_h.at[pl.ds(b, C)], bb.at[pl.ds(op, C)], semB)

        def drain():
            pltpu.make_async_copy(dt_h.at[pl.ds(0, C)],
                                  dtb.at[pl.ds(0, C)], semA).wait()
            pltpu.make_async_copy(dt_h.at[pl.ds(0, C)],
                                  dnb.at[pl.ds(0, C)], semA).wait()
            pltpu.make_async_copy(r_h.at[pl.ds(0, C)],
                                  rb.at[pl.ds(0, C)], semB).wait()
            pltpu.make_async_copy(r_h.at[pl.ds(0, C)],
                                  gb.at[pl.ds(0, C)], semB).wait()
            pltpu.make_async_copy(r_h.at[pl.ds(0, C)],
                                  bb.at[pl.ds(0, C)], semB).wait()

        def window_body(st):
            p, r, e_next, carry, par, accw, accr, accg, accb = st
            base = pl.multiple_of(
                jnp.minimum(p & ~jnp.int32(7), jnp.int32(n - C)), 8)
            wend = base + jnp.int32(C)
            drain()
            bnext = pl.multiple_of(
                jnp.minimum(wend, jnp.int32(n - C)), 8)
            issue(bnext, 1 - par)
            oc = pl.multiple_of(par * C, 8)
            opv = jnp.full((L,), par * (C + PAD), jnp.int32)

            @plsc.parallel_loop(0, C, step=L, unroll=8, carry=zero)
            def wc_end(off, wc):
                dtv = dtb[pl.ds(oc + off, L)]
                dnv = dnb[pl.ds(oc + off, L)]
                tv = dtv * dnv
                en = jnp.exp(-tv)
                albuf[pl.ds(off, L)] = 1.0 - en
                lx = jnp.float32(1e-7) * jnp.exp(tv) - tv
                inc = plsc.cumsum(lx)
                pbuf[pl.ds(off, L)] = (wc - lx) + inc
                return wc + inc[L - 1]
            plsc.store_scatter(pbuf, [jnp.full((L,), C, jnp.int32)],
                               jnp.full((L,), wc_end), mask=lane0)

            def seg_cond(sst):
                sp, sr, se = sst[0], sst[1], sst[2]
                return (sr < rpw) & ((sp < wend) | (se <= sp))

            def seg_body(sst):
                sp, sr, se, scarry, saw, sar, sag, sab = sst
                seg_end = jnp.minimum(se, wend)
                nsteps = (seg_end - sp + jnp.int32(L - 1)) // jnp.int32(L)
                cbase = scarry - sread(pbuf, sp - base)
                cb = jnp.full((L,), cbase, jnp.float32)

                loff = sp - base
                loffp = loff + par * (C + PAD)
                send = seg_end - sp

                @plsc.parallel_loop(0, nsteps * L, step=L,
                                    carry=(saw, sar, sag, sab))
                def accs(t, a):
                    taw, tar, tag, tab = a
                    valid = (t + iota) < send
                    pv = pbuf[pl.ds(loff + t, L)]
                    av = albuf[pl.ds(loff + t, L)]
                    w = jnp.where(valid, av * jnp.exp(cb + pv), zero)
                    rv = rb[pl.ds(loffp + t, L)]
                    gv = gb[pl.ds(loffp + t, L)]
                    bv = bb[pl.ds(loffp + t, L)]
                    return (taw + w, tar + rv * w, tag + gv * w,
                            tab + bv * w)

                saw, sar, sag, sab = accs
                scarry2 = cbase + sread(pbuf, seg_end - base)
                done = seg_end == se

                @pl.when(done)
                def _finalize():
                    rloc = jnp.full((L,), sr, jnp.int32)
                    plsc.store_scatter(ows, [rloc],
                                       jnp.full((L,), jnp.sum(saw)),
                                       mask=lane0)
                    plsc.store_scatter(obg, [rloc],
                                       jnp.exp(jnp.full((L,), scarry2)),
                                       mask=lane0)
                    rgbvals = jnp.where(
                        iota == 0, jnp.sum(sar),
                        jnp.where(iota == 1, jnp.sum(sag), jnp.sum(sab)))
                    plsc.store_scatter(orgb, [3 * rloc + iota], rgbvals,
                                       mask=iota < 3)

                keep = jnp.where(done, zero, jnp.float32(1.0))
                r2 = jnp.where(done, sr + 1, sr)
                e2 = jnp.where(done, sread(cuv, r2 + 1), se)
                return (seg_end, r2, e2, scarry2 * keep, saw * keep,
                        sar * keep, sag * keep, sab * keep)

            sp, sr, se, scarry, saw, sar, sag, sab = lax.while_loop(
                seg_cond, seg_body,
                (p, r, e_next, carry, accw, accr, accg, accb))
            return (sp, sr, se, scarry, 1 - par, saw, sar, sag, sab)

        p0 = sread(cuv, 0)
        issue(pl.multiple_of(
            jnp.minimum(p0 & ~jnp.int32(7), jnp.int32(n - C)), 8), 0)
        lax.while_loop(lambda st: st[1] < rpw, window_body,
                       (p0, jnp.int32(0), sread(cuv, 1),
                        zero, jnp.int32(0), fz, fz, fz, fz))
        drain()

        pltpu.sync_copy(ows, ws_h.at[pl.ds(r0, rpw)])
        pltpu.sync_copy(obg, bg_h.at[pl.ds(r0, rpw)])
        pltpu.sync_copy(orgb, rgbo_h.at[pl.ds(pl.multiple_of(r0 * 3, 8),
                                              rpw * 3)])

    return k


def kernel(samples_dt, density_samples, rgb_samples, cu_seqlens):
    n = samples_dt.shape[0]
    n_rays = cu_seqlens.shape[0] - 1
    rpw = n_rays // NW
    pad = NW * rpw - n_rays + CU_T
    cu_pad = jnp.concatenate(
        [cu_seqlens, jnp.full((pad,), n, jnp.int32)])
    ws, bg, prf = _build(n, n_rays)(
        samples_dt.reshape(n), density_samples.reshape(n),
        rgb_samples[:, 0], rgb_samples[:, 1], rgb_samples[:, 2], cu_pad)
    return (prf.reshape(n_rays, 3), ws.reshape(n_rays, 1),
            bg.reshape(n_rays, 1))


# speedup vs baseline: 1.0220x; 1.0220x over previous
import functools

import jax
import jax.numpy as jnp
from jax import lax
from jax.experimental import pallas as pl
from jax.experimental.pallas import tpu as pltpu
from jax.experimental.pallas import tpu_sc as plsc

NC = 2
NS = 16
NW = NC * NS
L = 16

C = 8192
PAD = 16
CU_T = 528


@functools.lru_cache(maxsize=None)
def _build(n, n_rays):
    rpw = n_rays // NW
    mesh = plsc.VectorSubcoreMesh(core_axis_name="c", subcore_axis_name="s",
                                  num_cores=NC, num_subcores=NS)

    @functools.partial(
        pl.kernel,
        mesh=mesh,
        compiler_params=pltpu.CompilerParams(needs_layout_passes=False),
        out_type=[
            jax.ShapeDtypeStruct((n_rays,), jnp.float32),
            jax.ShapeDtypeStruct((n_rays,), jnp.float32),
            jax.ShapeDtypeStruct((3 * n_rays,), jnp.float32),
        ],
        scratch_types=[
            pltpu.VMEM((2 * C,), jnp.float32),
            pltpu.VMEM((2 * C,), jnp.float32),
            pltpu.VMEM((2 * (C + PAD),), jnp.float32),
            pltpu.VMEM((2 * (C + PAD),), jnp.float32),
            pltpu.VMEM((2 * (C + PAD),), jnp.float32),
            pltpu.VMEM((C + PAD,), jnp.float32),
            pltpu.VMEM((C + PAD,), jnp.float32),
            pltpu.VMEM((CU_T,), jnp.int32),
            pltpu.VMEM((rpw,), jnp.float32),
            pltpu.VMEM((rpw,), jnp.float32),
            pltpu.VMEM((3 * rpw,), jnp.float32),
            pltpu.SemaphoreType.DMA,
            pltpu.SemaphoreType.DMA,
        ],
    )
    def k(dt_h, dens_h, r_h, g_h, b_h, cu_h, ws_h, bg_h, rgbo_h,
          dtb, dnb, rb, gb, bb, albuf, pbuf, cuv, ows, obg, orgb,
          semA, semB):
        wid = lax.axis_index("s") * NC + lax.axis_index("c")
        r0 = pl.multiple_of(wid * rpw, 8)
        pltpu.sync_copy(cu_h.at[pl.ds(r0, CU_T)], cuv)
        iota = lax.iota(jnp.int32, L)
        lane0 = iota == 0
        fz = jnp.zeros((L,), jnp.float32)
        zero = jnp.float32(0.0)

        def sread(ref, i):
            return plsc.load_gather(ref, [jnp.full((L,), i, jnp.int32)])[0]

        def issue(b, row):
            oc = pl.multiple_of(row * C, 8)
            op = pl.multiple_of(row * (C + PAD), 8)
            pltpu.async_copy(dt_h.at[pl.ds(b, C)], dtb.at[pl.ds(oc, C)],
                             semA)
            pltpu.async_copy(dens_h.at[pl.ds(b, C)], dnb.at[pl.ds(oc, C)],
                             semA)
            pltpu.async_copy(r_h.at[pl.ds(b, C)], rb.at[pl.ds(op, C)], semB)
            pltpu.async_copy(g_h.at[pl.ds(b, C)], gb.at[pl.ds(op, C)], semB)
            pltpu.async_copy(b_h.at[pl.ds(b, C)], bb.at[pl.ds(op, C)], semB)

        def drain():
            pltpu.make_async_copy(dt_h.at[pl.ds(0, C)],
                                  dtb.at[pl.ds(0, C)], semA).wait()
            pltpu.make_async_copy(dt_h.at[pl.ds(0, C)],
                                  dnb.at[pl.ds(0, C)], semA).wait()
            pltpu.make_async_copy(r_h.at[pl.ds(0, C)],
                                  rb.at[pl.ds(0, C)], semB).wait()
            pltpu.make_async_copy(r_h.at[pl.ds(0, C)],
                                  gb.at[pl.ds(0, C)], semB).wait()
            pltpu.make_async_copy(r_h.at[pl.ds(0, C)],
                                  bb.at[pl.ds(0, C)], semB).wait()

        def window_body(st):
            p, r, e_next, carry, par, accw, accr, accg, accb = st
            base = pl.multiple_of(
                jnp.minimum(p & ~jnp.int32(7), jnp.int32(n - C)), 8)
            wend = base + jnp.int32(C)
            drain()
            bnext = pl.multiple_of(
                jnp.minimum(wend, jnp.int32(n - C)), 8)
            issue(bnext, 1 - par)
            oc = pl.multiple_of(par * C, 8)
            opv = jnp.full((L,), par * (C + PAD), jnp.int32)

            @plsc.parallel_loop(0, C, step=L, unroll=8, carry=zero)
            def wc_end(off, wc):
                dtv = dtb[pl.ds(oc + off, L)]
                dnv = dnb[pl.ds(oc + off, L)]
                tv = dtv * dnv
                en = jnp.exp(-tv)
                albuf[pl.ds(off, L)] = 1.0 - en
                lx = jnp.float32(1e-7) * jnp.exp(tv) - tv
                inc = plsc.cumsum(lx)
                pbuf[pl.ds(off, L)] = (wc - lx) + inc
                return wc + inc[L - 1]
            plsc.store_scatter(pbuf, [jnp.full((L,), C, jnp.int32)],
                               jnp.full((L,), wc_end), mask=lane0)

            def seg_cond(sst):
                sp, sr, se = sst[0], sst[1], sst[2]
                return (sr < rpw) & ((sp < wend) | (se <= sp))

            def seg_body(sst):
                sp, sr, se, scarry, saw, sar, sag, sab = sst
                seg_end = jnp.minimum(se, wend)
                nsteps = (seg_end - sp + jnp.int32(L - 1)) // jnp.int32(L)
                cbase = scarry - sread(pbuf, sp - base)
                cb = jnp.full((L,), cbase, jnp.float32)

                loff = sp - base
                loffp = loff + par * (C + PAD)
                send = seg_end - sp

                @plsc.parallel_loop(0, nsteps * L, step=L, unroll=2,
                                    carry=(saw, sar, sag, sab))
                def accs(t, a):
                    taw, tar, tag, tab = a
                    valid = (t + iota) < send
                    pv = pbuf[pl.ds(loff + t, L)]
                    av = albuf[pl.ds(loff + t, L)]
                    w = jnp.where(valid, av * jnp.exp(cb + pv), zero)
                    rv = rb[pl.ds(loffp + t, L)]
                    gv = gb[pl.ds(loffp + t, L)]
                    bv = bb[pl.ds(loffp + t, L)]
                    return (taw + w, tar + rv * w, tag + gv * w,
                            tab + bv * w)

                saw, sar, sag, sab = accs
                scarry2 = cbase + sread(pbuf, seg_end - base)
                done = seg_end == se

                @pl.when(done)
                def _finalize():
                    rloc = jnp.full((L,), sr, jnp.int32)
                    plsc.store_scatter(ows, [rloc],
                                       jnp.full((L,), jnp.sum(saw)),
                                       mask=lane0)
                    plsc.store_scatter(obg, [rloc],
                                       jnp.exp(jnp.full((L,), scarry2)),
                                       mask=lane0)
                    rgbvals = jnp.where(
                        iota == 0, jnp.sum(sar),
                        jnp.where(iota == 1, jnp.sum(sag), jnp.sum(sab)))
                    plsc.store_scatter(orgb, [3 * rloc + iota], rgbvals,
                                       mask=iota < 3)

                keep = jnp.where(done, zero, jnp.float32(1.0))
                r2 = jnp.where(done, sr + 1, sr)
                e2 = jnp.where(done, sread(cuv, r2 + 1), se)
                return (seg_end, r2, e2, scarry2 * keep, saw * keep,
                        sar * keep, sag * keep, sab * keep)

            sp, sr, se, scarry, saw, sar, sag, sab = lax.while_loop(
                seg_cond, seg_body,
                (p, r, e_next, carry, accw, accr, accg, accb))
            return (sp, sr, se, scarry, 1 - par, saw, sar, sag, sab)

        p0 = sread(cuv, 0)
        issue(pl.multiple_of(
            jnp.minimum(p0 & ~jnp.int32(7), jnp.int32(n - C)), 8), 0)
        lax.while_loop(lambda st: st[1] < rpw, window_body,
                       (p0, jnp.int32(0), sread(cuv, 1),
                        zero, jnp.int32(0), fz, fz, fz, fz))
        drain()

        pltpu.sync_copy(ows, ws_h.at[pl.ds(r0, rpw)])
        pltpu.sync_copy(obg, bg_h.at[pl.ds(r0, rpw)])
        pltpu.sync_copy(orgb, rgbo_h.at[pl.ds(pl.multiple_of(r0 * 3, 8),
                                              rpw * 3)])

    return k


def kernel(samples_dt, density_samples, rgb_samples, cu_seqlens):
    n = samples_dt.shape[0]
    n_rays = cu_seqlens.shape[0] - 1
    rpw = n_rays // NW
    pad = NW * rpw - n_rays + CU_T
    cu_pad = jnp.concatenate(
        [cu_seqlens, jnp.full((pad,), n, jnp.int32)])
    ws, bg, prf = _build(n, n_rays)(
        samples_dt.reshape(n), density_samples.reshape(n),
        rgb_samples[:, 0], rgb_samples[:, 1], rgb_samples[:, 2], cu_pad)
    return (prf.reshape(n_rays, 3), ws.reshape(n_rays, 1),
            bg.reshape(n_rays, 1))
